# baseline (speedup 1.0000x reference)
import jax
import jax.numpy as jnp
from jax.experimental import pallas as pl
from jax.experimental.pallas import tpu as pltpu

R = 4
TOPK = 2
OUT = 64
C = 19
B = 2
H = 64
S = 4096
SE = S // H
HC = H * C
HB = 8
NJ = H // HB


def _fused_body(x_any, seg_ref, wbr_ref, wet_ref, be_ref, bg_ref,
                g_ref, y_ref, xbuf, gacc, sem):
    b = pl.program_id(0)
    j = pl.program_id(1)
    copies = [
        pltpu.make_async_copy(
            x_any.at[r, b, pl.ds(j * HB * SE, HB * SE)], xbuf.at[r], sem)
        for r in range(R)
    ]
    for cp in copies:
        cp.start()
    for cp in copies:
        cp.wait()

    @pl.when(j == 0)
    def _():
        gacc[...] = jnp.zeros_like(gacc)

    seg = seg_ref[...]
    for r in range(R):
        xflat = xbuf[r]
        xr = xflat.reshape(HB, SE, HC)
        y = jnp.dot(xflat, wet_ref[r], preferred_element_type=jnp.float32)
        y_ref[r, 0] = y + be_ref[r]
        for g in range(R):
            t1 = xr * wbr_ref[r, :, g]
            t2 = jnp.dot(t1.reshape(HB * SE, HC), seg,
                         preferred_element_type=jnp.float32)
            gacc[g] += jnp.sum(t2.reshape(HB, SE, SE), axis=0)

    @pl.when(j == NJ - 1)
    def _():
        g_ref[0] = gacc[...] + bg_ref[...]


def _combine_body(g_ref, y_ref, o_ref):
    g = g_ref[0]
    c = [g[:, i:i + 1] for i in range(R)]
    ranks = []
    for i in range(R):
        rk = jnp.zeros((S, 1), jnp.float32)
        for jx in range(R):
            if jx == i:
                continue
            beats = (c[jx] >= c[i]) if jx < i else (c[jx] > c[i])
            rk = rk + jnp.where(beats, 1.0, 0.0)
        ranks.append(rk)
    mx = jnp.maximum(jnp.maximum(c[0], c[1]), jnp.maximum(c[2], c[3]))
    es = [jnp.where(ranks[i] < 1.5, jnp.exp(c[i] - mx), 0.0)
          for i in range(R)]
    z = es[0] + es[1] + es[2] + es[3]
    out = (es[0] / z) * y_ref[0, 0]
    for i in range(1, R):
        out = out + (es[i] / z) * y_ref[i, 0]
    o_ref[0] = out


@jax.jit
def kernel(x, We, be, Wg, bg):
    xe = x.reshape(R, B, S, HC)
    wgt = Wg.reshape(R, H, R, C).transpose(2, 1, 0, 3)
    wbr = jnp.tile(wgt, (1, 1, 1, SE)).reshape(R, H, R, 1, HC)
    wet = We.transpose(0, 2, 1)
    be2 = be.reshape(R, 1, OUT)
    bg2 = bg.reshape(R, 1, 1)
    segm = (jnp.arange(HC)[:, None] // C ==
            jnp.arange(SE)[None, :]).astype(jnp.float32)

    gt, y = pl.pallas_call(
        _fused_body,
        grid=(B, NJ),
        in_specs=[
            pl.BlockSpec(memory_space=pl.ANY),
            pl.BlockSpec((HC, SE), lambda b, j: (0, 0)),
            pl.BlockSpec((R, HB, R, 1, HC), lambda b, j: (0, j, 0, 0, 0)),
            pl.BlockSpec((R, HC, OUT), lambda b, j: (0, 0, 0)),
            pl.BlockSpec((R, 1, OUT), lambda b, j: (0, 0, 0)),
            pl.BlockSpec((R, 1, 1), lambda b, j: (0, 0, 0)),
        ],
        out_specs=[
            pl.BlockSpec((1, R, SE, SE), lambda b, j: (b, 0, 0, 0)),
            pl.BlockSpec((R, 1, HB * SE, OUT), lambda b, j: (0, b, j, 0)),
        ],
        out_shape=[
            jax.ShapeDtypeStruct((B, R, SE, SE), jnp.float32),
            jax.ShapeDtypeStruct((R, B, S, OUT), jnp.float32),
        ],
        scratch_shapes=[
            pltpu.VMEM((R, HB * SE, HC), jnp.float32),
            pltpu.VMEM((R, SE, SE), jnp.float32),
            pltpu.SemaphoreType.DMA,
        ],
    )(xe, segm, wbr, wet, be2, bg2)

    g = gt.reshape(B, R, S).transpose(0, 2, 1)

    out = pl.pallas_call(
        _combine_body,
        grid=(B,),
        in_specs=[
            pl.BlockSpec((1, S, R), lambda b: (b, 0, 0)),
            pl.BlockSpec((R, 1, S, OUT), lambda b: (0, b, 0, 0)),
        ],
        out_specs=pl.BlockSpec((1, S, OUT), lambda b: (b, 0, 0)),
        out_shape=jax.ShapeDtypeStruct((B, S, OUT), jnp.float32),
    )(g, y)
    return out


# speedup vs baseline: 1.0557x; 1.0557x over previous
import jax
import jax.numpy as jnp
from jax.experimental import pallas as pl
from jax.experimental.pallas import tpu as pltpu

R = 4
TOPK = 2
OUT = 64
C = 19
B = 2
H = 64
S = 4096
SE = S // H
HC = H * C
HB = 8
NJ = H // HB


def _fused_body(xe_ref, seg_ref, wbr_ref, wet_ref, be_ref, bg_ref,
                g_ref, y_ref, gacc):
    j = pl.program_id(1)

    @pl.when(j == 0)
    def _():
        gacc[...] = jnp.zeros_like(gacc)

    seg = seg_ref[...]
    for r in range(R):
        xflat = xe_ref[r, 0]
        xr = xflat.reshape(HB, SE, HC)
        y = jnp.dot(xflat, wet_ref[r], preferred_element_type=jnp.float32)
        y_ref[r, 0] = y + be_ref[r]
        for g in range(R):
            t1 = xr * wbr_ref[r, :, g]
            t2 = jnp.dot(t1.reshape(HB * SE, HC), seg,
                         preferred_element_type=jnp.float32)
            gacc[g] += jnp.sum(t2.reshape(HB, SE, SE), axis=0)

    @pl.when(j == NJ - 1)
    def _():
        g_ref[0] = gacc[...] + bg_ref[...]


def _combine_body(g_ref, y_ref, o_ref):
    g = g_ref[0]
    c = [g[:, i:i + 1] for i in range(R)]
    ranks = []
    for i in range(R):
        rk = jnp.zeros((S, 1), jnp.float32)
        for jx in range(R):
            if jx == i:
                continue
            beats = (c[jx] >= c[i]) if jx < i else (c[jx] > c[i])
            rk = rk + jnp.where(beats, 1.0, 0.0)
        ranks.append(rk)
    mx = jnp.maximum(jnp.maximum(c[0], c[1]), jnp.maximum(c[2], c[3]))
    es = [jnp.where(ranks[i] < 1.5, jnp.exp(c[i] - mx), 0.0)
          for i in range(R)]
    z = es[0] + es[1] + es[2] + es[3]
    out = (es[0] / z) * y_ref[0, 0]
    for i in range(1, R):
        out = out + (es[i] / z) * y_ref[i, 0]
    o_ref[0] = out


@jax.jit
def kernel(x, We, be, Wg, bg):
    xe = x.reshape(R, B, S, HC)
    wgt = Wg.reshape(R, H, R, C).transpose(2, 1, 0, 3)
    wbr = jnp.tile(wgt, (1, 1, 1, SE)).reshape(R, H, R, 1, HC)
    wet = We.transpose(0, 2, 1)
    be2 = be.reshape(R, 1, OUT)
    bg2 = bg.reshape(R, 1, 1)
    segm = (jnp.arange(HC)[:, None] // C ==
            jnp.arange(SE)[None, :]).astype(jnp.float32)

    gt, y = pl.pallas_call(
        _fused_body,
        grid=(B, NJ),
        in_specs=[
            pl.BlockSpec((R, 1, HB * SE, HC), lambda b, j: (0, b, j, 0)),
            pl.BlockSpec((HC, SE), lambda b, j: (0, 0)),
            pl.BlockSpec((R, HB, R, 1, HC), lambda b, j: (0, j, 0, 0, 0)),
            pl.BlockSpec((R, HC, OUT), lambda b, j: (0, 0, 0)),
            pl.BlockSpec((R, 1, OUT), lambda b, j: (0, 0, 0)),
            pl.BlockSpec((R, 1, 1), lambda b, j: (0, 0, 0)),
        ],
        out_specs=[
            pl.BlockSpec((1, R, SE, SE), lambda b, j: (b, 0, 0, 0)),
            pl.BlockSpec((R, 1, HB * SE, OUT), lambda b, j: (0, b, j, 0)),
        ],
        out_shape=[
            jax.ShapeDtypeStruct((B, R, SE, SE), jnp.float32),
            jax.ShapeDtypeStruct((R, B, S, OUT), jnp.float32),
        ],
        scratch_shapes=[
            pltpu.VMEM((R, SE, SE), jnp.float32),
        ],
    )(xe, segm, wbr, wet, be2, bg2)

    g = gt.reshape(B, R, S).transpose(0, 2, 1)

    out = pl.pallas_call(
        _combine_body,
        grid=(B,),
        in_specs=[
            pl.BlockSpec((1, S, R), lambda b: (b, 0, 0)),
            pl.BlockSpec((R, 1, S, OUT), lambda b: (0, b, 0, 0)),
        ],
        out_specs=pl.BlockSpec((1, S, OUT), lambda b: (b, 0, 0)),
        out_shape=jax.ShapeDtypeStruct((B, S, OUT), jnp.float32),
    )(g, y)
    return out
